# baseline (speedup 1.0000x reference)
import functools

import jax
import jax.numpy as jnp
from jax.experimental import pallas as pl
from jax.experimental.pallas import tpu as pltpu
from jax.experimental.pallas import tpu_sc as plsc

_A = 12.5
_RH = 0.65
_CA = 420.0
_COEF = _A * _RH / _CA

_NUM_FGS = 1


def _scs_body(fgs_hbm, gs0_hbm, a1_hbm, out_hbm, idx_s, g_s, a_s, o_s, sem):
    c1 = pltpu.async_copy(fgs_hbm, idx_s, sem)
    c2 = pltpu.async_copy(gs0_hbm, g_s, sem)
    c3 = pltpu.async_copy(a1_hbm, a_s, sem)
    c1.wait()
    c2.wait()
    c3.wait()
    i = idx_s[0]
    o_s[0] = g_s[i] + a_s[i] * _COEF
    pltpu.sync_copy(o_s, out_hbm)


def kernel(gs0, a1, FGs):
    fgs = FGs.astype(jnp.int32)
    mesh = plsc.ScalarSubcoreMesh(axis_name="c", num_cores=1)
    run = functools.partial(
        pl.kernel,
        mesh=mesh,
        out_type=jax.ShapeDtypeStruct((_NUM_FGS,), jnp.float32),
        scratch_types=[
            pltpu.SMEM((_NUM_FGS,), jnp.int32),
            pltpu.SMEM((_NUM_FGS,), jnp.float32),
            pltpu.SMEM((_NUM_FGS,), jnp.float32),
            pltpu.SMEM((_NUM_FGS,), jnp.float32),
            pltpu.SemaphoreType.DMA,
        ],
    )(_scs_body)
    return run(fgs, gs0, a1)
